# baseline (speedup 1.0000x reference)
import functools

import jax
import jax.numpy as jnp
from jax import lax
from jax.experimental import pallas as pl
from jax.experimental.pallas import tpu as pltpu
from jax.experimental.pallas import tpu_sc as plsc

N_CLASSES = 100000
EMBED_DIM = 64
BATCH = 16384

_info = plsc.get_sparse_core_info()
_NC, _NS = _info.num_cores, _info.num_subcores
_NW = _NC * _NS
_B_PER_W = BATCH // _NW
_G = _B_PER_W // 16

_W = 512
_TSTEPS = -(-N_CLASSES // (2 * _W))

_mesh = plsc.VectorSubcoreMesh(core_axis_name="c", subcore_axis_name="s")


def _tc_transpose_body(in0_ref, in1_ref, out_ref):
    out_ref[:, 0:EMBED_DIM] = jnp.swapaxes(in0_ref[...], 0, 1)
    out_ref[:, EMBED_DIM:] = jnp.swapaxes(in1_ref[...], 0, 1)


def _tc_transpose(embT):
    return pl.pallas_call(
        _tc_transpose_body,
        grid=(_TSTEPS,),
        in_specs=[
            pl.BlockSpec((EMBED_DIM, _W), lambda i: (0, 2 * i)),
            pl.BlockSpec((EMBED_DIM, _W), lambda i: (0, 2 * i + 1)),
        ],
        out_specs=pl.BlockSpec((_W, 2 * EMBED_DIM), lambda i: (i, 0)),
        out_shape=jax.ShapeDtypeStruct((_TSTEPS * _W, 2 * EMBED_DIM), jnp.float32),
    )(embT, embT)


@functools.partial(
    pl.kernel,
    mesh=_mesh,
    out_type=jax.ShapeDtypeStruct((EMBED_DIM, BATCH), jnp.float32),
    scratch_types=[
        pltpu.VMEM((_B_PER_W,), jnp.int32),
        pltpu.VMEM((_B_PER_W,), jnp.int32),
        pltpu.VMEM((_B_PER_W, 2 * EMBED_DIM), jnp.float32),
        pltpu.VMEM((EMBED_DIM, _B_PER_W), jnp.float32),
        pltpu.SemaphoreType.DMA,
    ],
    compiler_params=pltpu.CompilerParams(
        use_tc_tiling_on_sc=True, needs_layout_passes=False
    ),
)
def _embed_lookup(idx_hbm, table2_hbm, outT_hbm, idx_v, pidx_v, buf_v, outT_v, sem):
    wid = lax.axis_index("s") * _NC + lax.axis_index("c")
    base = wid * _B_PER_W
    lane = lax.iota(jnp.int32, 16)

    pltpu.sync_copy(idx_hbm.at[pl.ds(base, _B_PER_W)], idx_v)
    for u in range(_G):
        v = idx_v[pl.ds(u * 16, 16)]
        pidx_v[pl.ds(u * 16, 16)] = ((v >> 10) << 9) + (v & (_W - 1))
    pltpu.async_copy(table2_hbm.at[pidx_v], buf_v, sem).wait()

    def body(g, carry):
        i_v = g * 16 + lane
        vg = idx_v[pl.ds(g * 16, 16)]
        half_v = ((vg >> 9) & 1) * EMBED_DIM
        col_v = g * 16 + lane
        for k in range(16):
            perm = (lane + k) & 15
            for jb in range(EMBED_DIM // 16):
                j_v = jb * 16 + perm
                vals = plsc.load_gather(buf_v, [i_v, half_v + j_v])
                plsc.store_scatter(outT_v, [j_v, col_v], vals)
        return carry

    lax.fori_loop(0, _G, body, 0)
    pltpu.sync_copy(outT_v, outT_hbm.at[:, pl.ds(base, _B_PER_W)])


def kernel(cls, embed):
    table2 = _tc_transpose(embed.T)
    outT = _embed_lookup(cls.astype(jnp.int32), table2)
    return outT.T


# speedup vs baseline: 1.0584x; 1.0584x over previous
import functools

import jax
import jax.numpy as jnp
from jax import lax
from jax.experimental import pallas as pl
from jax.experimental.pallas import tpu as pltpu
from jax.experimental.pallas import tpu_sc as plsc

N_CLASSES = 100000
EMBED_DIM = 64
BATCH = 16384

_info = plsc.get_sparse_core_info()
_NC, _NS = _info.num_cores, _info.num_subcores
_NW = _NC * _NS
_B_PER_W = BATCH // _NW
_G = _B_PER_W // 16

_W = 512
_TSTEPS = -(-N_CLASSES // (2 * _W))

_mesh = plsc.VectorSubcoreMesh(core_axis_name="c", subcore_axis_name="s")


def _tc_transpose_body(in0_ref, in1_ref, out_ref):
    x = jnp.concatenate([in0_ref[...], in1_ref[...]], axis=0)
    out_ref[...] = jnp.swapaxes(x, 0, 1)


def _tc_transpose(embT):
    return pl.pallas_call(
        _tc_transpose_body,
        grid=(_TSTEPS,),
        in_specs=[
            pl.BlockSpec((EMBED_DIM, _W), lambda i: (0, 2 * i)),
            pl.BlockSpec((EMBED_DIM, _W), lambda i: (0, 2 * i + 1)),
        ],
        out_specs=pl.BlockSpec((_W, 2 * EMBED_DIM), lambda i: (i, 0)),
        out_shape=jax.ShapeDtypeStruct((_TSTEPS * _W, 2 * EMBED_DIM), jnp.float32),
    )(embT, embT)


@functools.partial(
    pl.kernel,
    mesh=_mesh,
    out_type=jax.ShapeDtypeStruct((EMBED_DIM, BATCH), jnp.float32),
    scratch_types=[
        pltpu.VMEM((_B_PER_W,), jnp.int32),
        pltpu.VMEM((_B_PER_W,), jnp.int32),
        pltpu.VMEM((_B_PER_W, 2 * EMBED_DIM), jnp.float32),
        pltpu.VMEM((EMBED_DIM, _B_PER_W), jnp.float32),
        pltpu.SemaphoreType.DMA,
    ],
    compiler_params=pltpu.CompilerParams(
        use_tc_tiling_on_sc=True, needs_layout_passes=False
    ),
)
def _embed_lookup(idx_hbm, table2_hbm, outT_hbm, idx_v, pidx_v, buf_v, outT_v, sem):
    wid = lax.axis_index("s") * _NC + lax.axis_index("c")
    base = wid * _B_PER_W
    lane = lax.iota(jnp.int32, 16)

    pltpu.sync_copy(idx_hbm.at[pl.ds(base, _B_PER_W)], idx_v)
    for u in range(_G):
        v = idx_v[pl.ds(u * 16, 16)]
        pidx_v[pl.ds(u * 16, 16)] = ((v >> 10) << 9) + (v & (_W - 1))
    pltpu.async_copy(table2_hbm.at[pidx_v], buf_v, sem).wait()

    def body(g, carry):
        i_v = g * 16 + lane
        vg = idx_v[pl.ds(g * 16, 16)]
        half_v = ((vg >> 9) & 1) * EMBED_DIM
        col_v = g * 16 + lane
        for k in range(16):
            perm = (lane + k) & 15
            for jb in range(EMBED_DIM // 16):
                j_v = jb * 16 + perm
                vals = plsc.load_gather(buf_v, [i_v, half_v + j_v])
                plsc.store_scatter(outT_v, [j_v, col_v], vals)
        return carry

    lax.fori_loop(0, _G, body, 0)
    pltpu.sync_copy(outT_v, outT_hbm.at[:, pl.ds(base, _B_PER_W)])


def kernel(cls, embed):
    table2 = _tc_transpose(embed.T)
    outT = _embed_lookup(cls.astype(jnp.int32), table2)
    return outT.T


# speedup vs baseline: 1.0713x; 1.0122x over previous
import functools

import jax
import jax.numpy as jnp
from jax import lax
from jax.experimental import pallas as pl
from jax.experimental.pallas import tpu as pltpu
from jax.experimental.pallas import tpu_sc as plsc

N_CLASSES = 100000
EMBED_DIM = 64
BATCH = 16384

_info = plsc.get_sparse_core_info()
_NC, _NS = _info.num_cores, _info.num_subcores
_NW = _NC * _NS
_B_PER_W = BATCH // _NW
_G = _B_PER_W // 16

_mesh = plsc.VectorSubcoreMesh(core_axis_name="c", subcore_axis_name="s")


@functools.partial(
    pl.kernel,
    mesh=_mesh,
    out_type=jax.ShapeDtypeStruct((EMBED_DIM, BATCH), jnp.float32),
    scratch_types=[
        pltpu.VMEM((_B_PER_W,), jnp.int32),
        pltpu.VMEM((_B_PER_W,), jnp.int32),
        pltpu.VMEM((_B_PER_W, 2 * EMBED_DIM), jnp.float32),
        pltpu.VMEM((EMBED_DIM, _B_PER_W), jnp.float32),
        pltpu.SemaphoreType.DMA,
    ],
    compiler_params=pltpu.CompilerParams(
        use_tc_tiling_on_sc=True, needs_layout_passes=False
    ),
)
def _embed_lookup(idx_hbm, table2_hbm, outT_hbm, idx_v, pidx_v, buf_v, outT_v, sem):
    wid = lax.axis_index("s") * _NC + lax.axis_index("c")
    base = wid * _B_PER_W
    lane = lax.iota(jnp.int32, 16)

    pltpu.sync_copy(idx_hbm.at[pl.ds(base, _B_PER_W)], idx_v)
    for u in range(_G):
        pidx_v[pl.ds(u * 16, 16)] = idx_v[pl.ds(u * 16, 16)] >> 1
    pltpu.async_copy(table2_hbm.at[pidx_v], buf_v, sem).wait()

    def body(g, carry):
        i_v = g * 16 + lane
        half_v = (idx_v[pl.ds(g * 16, 16)] & 1) * EMBED_DIM
        col_v = g * 16 + lane
        for k in range(16):
            perm = (lane + k) & 15
            for jb in range(EMBED_DIM // 16):
                j_v = jb * 16 + perm
                vals = plsc.load_gather(buf_v, [i_v, half_v + j_v])
                plsc.store_scatter(outT_v, [j_v, col_v], vals)
        return carry

    lax.fori_loop(0, _G, body, 0)
    pltpu.sync_copy(outT_v, outT_hbm.at[:, pl.ds(base, _B_PER_W)])


def kernel(cls, embed):
    table2 = embed.reshape(N_CLASSES // 2, 2 * EMBED_DIM)
    outT = _embed_lookup(cls.astype(jnp.int32), table2)
    return outT.T
